# baseline (speedup 1.0000x reference)
import functools

import jax
import jax.numpy as jnp
from jax.experimental import pallas as pl
from jax.experimental.pallas import tpu as pltpu

_BM = 1024
_WC = 256


def _dot_tb(xb, wb):
    return jax.lax.dot_general(
        xb, wb,
        dimension_numbers=(((1,), (1,)), ((), ())),
        preferred_element_type=jnp.float32)


def _fused_kernel(w_hbm, x_ref, b_ref, o_ref, wb_ref, wch_ref, sem):
    n_total = wb_ref.shape[0]
    n_chunks = n_total // _WC

    def _start(c):
        pltpu.make_async_copy(
            w_hbm.at[pl.ds(c * _WC, _WC), :], wch_ref.at[c % 2],
            sem.at[c % 2]).start()

    def _wait(c):
        pltpu.make_async_copy(
            w_hbm.at[pl.ds(c * _WC, _WC), :], wch_ref.at[c % 2],
            sem.at[c % 2]).wait()

    @pl.when(pl.program_id(1) == 0)
    def _():
        xb = x_ref[...].astype(jnp.bfloat16)
        _start(0)
        _start(1)
        for c in range(n_chunks):
            _wait(c)
            wbc = wch_ref[c % 2].astype(jnp.bfloat16)
            wb_ref[c * _WC:(c + 1) * _WC, :] = wbc
            if c + 2 < n_chunks:
                _start(c + 2)
            o_ref[:, c * _WC:(c + 1) * _WC] = (
                _dot_tb(xb, wbc) + b_ref[:, c * _WC:(c + 1) * _WC])

    @pl.when(pl.program_id(1) > 0)
    def _():
        xb = x_ref[...].astype(jnp.bfloat16)
        o_ref[...] = _dot_tb(xb, wb_ref[...]) + b_ref[...]


@functools.partial(jax.jit, static_argnames=("bm",))
def _forward(x, w, b, *, bm):
    M, K = x.shape
    N = w.shape[0]
    b_row = b.reshape(1, N)
    steps = M // bm // 2
    grid = (2, steps)
    out = pl.pallas_call(
        _fused_kernel,
        out_shape=jax.ShapeDtypeStruct((M, N), jnp.float32),
        grid=grid,
        in_specs=[
            pl.BlockSpec(memory_space=pltpu.MemorySpace.HBM),
            pl.BlockSpec((bm, K), lambda i, j: (i * steps + j, 0)),
            pl.BlockSpec((1, N), lambda i, j: (0, 0)),
        ],
        out_specs=pl.BlockSpec((bm, N), lambda i, j: (i * steps + j, 0)),
        scratch_shapes=[
            pltpu.VMEM((N, K), jnp.bfloat16),
            pltpu.VMEM((2, _WC, K), jnp.float32),
            pltpu.SemaphoreType.DMA((2,)),
        ],
        compiler_params=pltpu.CompilerParams(
            dimension_semantics=("parallel", "arbitrary"),
            vmem_limit_bytes=62 * 1024 * 1024),
        cost_estimate=pl.CostEstimate(
            flops=2 * M * N * K,
            bytes_accessed=4 * M * K + 4 * K * N + 4 * M * N,
            transcendentals=0),
    )(w, x, b_row)
    return out


def kernel(x, w, b):
    bm = _BM if x.shape[0] % (2 * _BM) == 0 else 8
    return _forward(x, w, b, bm=bm)


# speedup vs baseline: 1.2841x; 1.2841x over previous
import functools

import jax
import jax.numpy as jnp
from jax.experimental import pallas as pl
from jax.experimental.pallas import tpu as pltpu

_BM = 1024


def _fused_kernel(w_ref, x_ref, b_ref, o_ref, wb_ref):
    @pl.when(pl.program_id(1) == 0)
    def _():
        wb_ref[...] = w_ref[...].astype(jnp.bfloat16)

    xb = x_ref[...].astype(jnp.bfloat16)
    acc = jax.lax.dot_general(
        xb, wb_ref[...],
        dimension_numbers=(((1,), (1,)), ((), ())),
        preferred_element_type=jnp.float32)
    o_ref[...] = acc + b_ref[...]


@functools.partial(jax.jit, static_argnames=("bm",))
def _forward(x, w, b, *, bm):
    M, K = x.shape
    N = w.shape[0]
    b_row = b.reshape(1, N)
    steps = M // bm // 2
    grid = (2, steps)
    out = pl.pallas_call(
        _fused_kernel,
        out_shape=jax.ShapeDtypeStruct((M, N), jnp.float32),
        grid=grid,
        in_specs=[
            pl.BlockSpec((N, K), lambda i, j: (0, 0)),
            pl.BlockSpec((bm, K), lambda i, j: (i * steps + j, 0)),
            pl.BlockSpec((1, N), lambda i, j: (0, 0)),
        ],
        out_specs=pl.BlockSpec((bm, N), lambda i, j: (i * steps + j, 0)),
        scratch_shapes=[pltpu.VMEM((N, K), jnp.bfloat16)],
        compiler_params=pltpu.CompilerParams(
            dimension_semantics=("parallel", "arbitrary"),
            vmem_limit_bytes=62 * 1024 * 1024),
        cost_estimate=pl.CostEstimate(
            flops=2 * M * N * K,
            bytes_accessed=4 * M * K + 4 * K * N + 4 * M * N,
            transcendentals=0),
    )(w, x, b_row)
    return out


def kernel(x, w, b):
    bm = _BM if x.shape[0] % (2 * _BM) == 0 else 8
    return _forward(x, w, b, bm=bm)


# speedup vs baseline: 1.3032x; 1.0149x over previous
import functools

import jax
import jax.numpy as jnp
from jax.experimental import pallas as pl
from jax.experimental.pallas import tpu as pltpu

_BM = 1024


def _linear_kernel(w_ref, x_ref, b_ref, o_ref):
    acc = jax.lax.dot_general(
        x_ref[...], w_ref[...],
        dimension_numbers=(((1,), (1,)), ((), ())),
        preferred_element_type=jnp.float32)
    o_ref[...] = acc + b_ref[...]


@functools.partial(jax.jit, static_argnames=("bm",))
def _forward(x, w, b, *, bm):
    M, K = x.shape
    N = w.shape[0]
    b_row = b.reshape(1, N)
    steps = M // bm // 2
    grid = (2, steps)
    out = pl.pallas_call(
        _linear_kernel,
        out_shape=jax.ShapeDtypeStruct((M, N), jnp.float32),
        grid=grid,
        in_specs=[
            pl.BlockSpec((N, K), lambda i, j: (0, 0)),
            pl.BlockSpec((bm, K), lambda i, j: (i * steps + j, 0)),
            pl.BlockSpec((1, N), lambda i, j: (0, 0)),
        ],
        out_specs=pl.BlockSpec((bm, N), lambda i, j: (i * steps + j, 0)),
        compiler_params=pltpu.CompilerParams(
            dimension_semantics=("parallel", "arbitrary"),
            vmem_limit_bytes=62 * 1024 * 1024),
        cost_estimate=pl.CostEstimate(
            flops=2 * M * N * K,
            bytes_accessed=4 * M * K + 4 * K * N + 4 * M * N,
            transcendentals=0),
    )(w, x, b_row)
    return out


def kernel(x, w, b):
    bm = _BM if x.shape[0] % (2 * _BM) == 0 else 8
    return _forward(x, w, b, bm=bm)
